# baseline (speedup 1.0000x reference)
import functools

import jax
import jax.numpy as jnp
from jax import lax
from jax.experimental import pallas as pl
from jax.experimental.pallas import tpu as pltpu
from jax.experimental.pallas import tpu_sc as plsc

_B = 16384
_D = 32
_NC = 2
_NS = 16
_NW = _NC * _NS
_PW = _B // _NW
_CH = 128
_NCH = _PW // _CH


@functools.partial(
    pl.kernel,
    mesh=plsc.VectorSubcoreMesh(core_axis_name="c", subcore_axis_name="s"),
    out_type=jax.ShapeDtypeStruct((_B, 2, _D), jnp.float32),
    compiler_params=pltpu.CompilerParams(use_tc_tiling_on_sc=False),
    scratch_types=[
        pltpu.VMEM((_NCH, _CH), jnp.int32),
        pltpu.VMEM((_NCH, _CH), jnp.int32),
        pltpu.VMEM((_PW, _D), jnp.float32),
        pltpu.VMEM((_PW, _D), jnp.float32),
        pltpu.SemaphoreType.DMA,
    ],
)
def _edge_gather(src_hbm, dst_hbm, table_hbm, out_hbm,
                 idx_s, idx_d, rows_s, rows_d, sem):
    wid = lax.axis_index("s") * _NC + lax.axis_index("c")
    pltpu.sync_copy(src_hbm.at[wid], idx_s)
    pltpu.sync_copy(dst_hbm.at[wid], idx_d)

    copies = []
    for j in range(_NCH):
        copies.append(pltpu.async_copy(
            table_hbm.at[idx_s.at[j]], rows_s.at[pl.ds(j * _CH, _CH)], sem))
        copies.append(pltpu.async_copy(
            table_hbm.at[idx_d.at[j]], rows_d.at[pl.ds(j * _CH, _CH)], sem))
    for c in copies:
        c.wait()

    base = wid * _PW
    pltpu.sync_copy(rows_s, out_hbm.at[pl.ds(base, _PW), 0])
    pltpu.sync_copy(rows_d, out_hbm.at[pl.ds(base, _PW), 1])


def kernel(source_node_input, destination_node_input, table):
    src = source_node_input.reshape(_NW, _NCH, _CH)
    dst = destination_node_input.reshape(_NW, _NCH, _CH)
    rows = _edge_gather(src, dst, table)
    return rows.reshape(_B, 2 * _D)
